# Initial kernel scaffold; baseline (speedup 1.0000x reference)
import jax
import jax.numpy as jnp
from jax.experimental import pallas as pl


def kernel(kind_ids, other_feats, edge_index, cand_u, cand_v, kind_table, W0, b0, W1, b1, W2, b2, Wg, bg, Wc1, bc1, Wc2, bc2):
    raise NotImplementedError("write your pallas kernel here")


# speedup vs baseline: 12.8892x; 12.8892x over previous
import functools

import jax
import jax.numpy as jnp
from jax import lax
from jax.experimental import pallas as pl
from jax.experimental.pallas import tpu as pltpu
from jax.experimental.pallas import tpu_sc as plsc

N = 50000
E = 800000
C = 4096
H = 64
NPAD = 50048
RPT = 3128
EB = 1000
R = 400

_mesh = plsc.VectorSubcoreMesh(core_axis_name="c", subcore_axis_name="s")

_f32 = jnp.float32
_i32 = jnp.int32


def _fill_const_2d(buf, nrows, width, val):
    vec = jnp.full((16,), val, _f32)

    def body(i, carry):
        for w0 in range(0, width, 16):
            buf[i, pl.ds(w0, 16)] = vec
        return carry

    lax.fori_loop(0, nrows, body, 0)


def _fill_const_1d(buf, n, val):
    vec = jnp.full((16,), val, _f32)

    def body(i, carry):
        buf[pl.ds(i * 16, 16)] = vec
        return carry

    lax.fori_loop(0, n // 16, body, 0)
    if n % 16:
        buf[pl.ds(n - 16, 16)] = vec


def _zero_rows_2d(agg, zbuf, row0):
    for off in (0, 1024, 2048):
        pltpu.sync_copy(zbuf.at[:, :], agg.at[pl.ds(row0 + off, 1024), :])
    pltpu.sync_copy(zbuf.at[pl.ds(0, 56), :], agg.at[pl.ds(row0 + 3072, 56), :])


def _zero_rows_1d(agg, zbuf, row0):
    for off in (0, 1024, 2048):
        pltpu.sync_copy(zbuf.at[pl.ds(0, 1024)], agg.at[pl.ds(row0 + off, 1024)])
    pltpu.sync_copy(zbuf.at[pl.ds(0, 56)], agg.at[pl.ds(row0 + 3072, 56)])


_CHUNKS = ((0, 1024), (1024, 1024), (2048, 1024), (3072, 56))


def _writeout_2d(agg, buf, out, c, row0):
    for off, sz in _CHUNKS:
        pltpu.sync_copy(agg.at[pl.ds(row0 + off, sz), :], buf.at[pl.ds(0, sz), :])
        pltpu.sync_copy(buf.at[pl.ds(0, sz), :], out.at[c, pl.ds(row0 + off, sz), :])


def _segsum_edges(src, dst, z, agg, src_v, dst_v, rows_v, sem, base, nblocks):
    def body(i, carry):
        b = base + i * EB
        pltpu.sync_copy(src.at[pl.ds(b, EB)], src_v)
        pltpu.sync_copy(dst.at[pl.ds(b, EB)], dst_v)
        pltpu.async_copy(z.at[src_v], rows_v, sem).wait()
        pltpu.sync_copy(rows_v, agg.at[dst_v], add=True)
        return carry

    lax.fori_loop(0, nblocks, body, 0)


@functools.partial(
    pl.kernel,
    out_type=jax.ShapeDtypeStruct((2, NPAD), _f32),
    mesh=_mesh,
    compiler_params=pltpu.CompilerParams(use_tc_tiling_on_sc=False),
    scratch_types=[
        pltpu.VMEM((EB,), _i32),
        pltpu.VMEM((EB,), _f32),
        pltpu.VMEM((1024,), _f32),
        pltpu.VMEM_SHARED((NPAD,), _f32),
    ],
)
def _sc_degree(dst, out, dst_v, ones_v, zbuf, deg_sh):
    c = lax.axis_index("c")
    s = lax.axis_index("s")
    row0 = s * RPT
    _fill_const_1d(zbuf, 1024, 0.0)
    _fill_const_1d(ones_v, EB, 1.0)
    _zero_rows_1d(deg_sh, zbuf, row0)
    plsc.subcore_barrier()

    base = c * 400000 + s * 25000

    def body(i, carry):
        pltpu.sync_copy(dst.at[pl.ds(base + i * EB, EB)], dst_v)
        pltpu.sync_copy(ones_v.at[pl.ds(0, EB)], deg_sh.at[dst_v], add=True)
        return carry

    lax.fori_loop(0, 25, body, 0)
    plsc.subcore_barrier()

    for off, sz in _CHUNKS:
        pltpu.sync_copy(deg_sh.at[pl.ds(row0 + off, sz)], zbuf.at[pl.ds(0, sz)])
        pltpu.sync_copy(zbuf.at[pl.ds(0, sz)], out.at[c, pl.ds(row0 + off, sz)])


@functools.partial(
    pl.kernel,
    out_type=jax.ShapeDtypeStruct((2, NPAD, 16), _f32),
    mesh=_mesh,
    compiler_params=pltpu.CompilerParams(use_tc_tiling_on_sc=False),
    scratch_types=[
        pltpu.VMEM((EB,), _i32),
        pltpu.VMEM((EB,), _i32),
        pltpu.VMEM((EB, 16), _f32),
        pltpu.VMEM((1024, 16), _f32),
        pltpu.VMEM_SHARED((NPAD, 16), _f32),
        pltpu.SemaphoreType.DMA,
    ],
)
def _sc_segsum16(src, dst, z0, out, src_v, dst_v, rows_v, zbuf, agg, sem):
    c = lax.axis_index("c")
    s = lax.axis_index("s")
    row0 = s * RPT
    _fill_const_2d(zbuf, 1024, 16, 0.0)
    _zero_rows_2d(agg, zbuf, row0)
    plsc.subcore_barrier()

    base = c * 400000 + s * 25000
    _segsum_edges(src, dst, z0, agg, src_v, dst_v, rows_v, sem, base, 25)
    plsc.subcore_barrier()
    _writeout_2d(agg, zbuf, out, c, row0)


@functools.partial(
    pl.kernel,
    out_type=jax.ShapeDtypeStruct((4, NPAD, 16), _f32),
    mesh=_mesh,
    compiler_params=pltpu.CompilerParams(use_tc_tiling_on_sc=False),
    scratch_types=[
        pltpu.VMEM((EB,), _i32),
        pltpu.VMEM((EB,), _i32),
        pltpu.VMEM((EB, 16), _f32),
        pltpu.VMEM((1024, 16), _f32),
        pltpu.VMEM((1024, 16), _f32),
        pltpu.VMEM_SHARED((NPAD, 16), _f32),
        pltpu.SemaphoreType.DMA,
    ],
)
def _sc_segsum64(src, dst, z4, out, src_v, dst_v, rows_v, zbuf, wbuf, agg, sem):
    c = lax.axis_index("c")
    s = lax.axis_index("s")
    row0 = s * RPT
    _fill_const_2d(zbuf, 1024, 16, 0.0)
    _zero_rows_2d(agg, zbuf, row0)

    base = s * 50000
    for p in range(2):
        chunk = 2 * c + p
        plsc.subcore_barrier()
        _segsum_edges(src, dst, z4.at[chunk], agg, src_v, dst_v, rows_v, sem,
                      base, 50)
        plsc.subcore_barrier()
        for off, sz in _CHUNKS:
            pltpu.sync_copy(agg.at[pl.ds(row0 + off, sz), :],
                            wbuf.at[pl.ds(0, sz), :])
            pltpu.sync_copy(wbuf.at[pl.ds(0, sz), :],
                            out.at[chunk, pl.ds(row0 + off, sz), :])
        if p == 0:
            _zero_rows_2d(agg, zbuf, row0)


@functools.partial(
    pl.kernel,
    out_type=jax.ShapeDtypeStruct((2, C, H), _f32),
    mesh=_mesh,
    compiler_params=pltpu.CompilerParams(use_tc_tiling_on_sc=False),
    scratch_types=[
        pltpu.VMEM((256,), _i32),
        pltpu.VMEM((256, H), _f32),
        pltpu.SemaphoreType.DMA,
    ],
)
def _sc_cand_gather(h, cuv, out, idx_v, rows_v, sem):
    c = lax.axis_index("c")
    s = lax.axis_index("s")
    base = s * 256
    pltpu.sync_copy(cuv.at[c, pl.ds(base, 256)], idx_v)
    pltpu.async_copy(h.at[idx_v], rows_v, sem).wait()
    pltpu.sync_copy(rows_v, out.at[c, pl.ds(base, 256), :])


def _encode_body(kid, other, d0, d1, ktab, z0_ref, nrm_ref):
    n = lax.rsqrt(d0[...] + d1[...] + 1.0)
    oh = (kid[...] == lax.broadcasted_iota(_i32, (R, 8), 1)).astype(_f32)
    emb = jnp.dot(oh, ktab[...], preferred_element_type=_f32, precision=lax.Precision.HIGHEST)
    x = jnp.concatenate([emb, other[...], jnp.zeros((R, 3), _f32)], axis=1)
    z0_ref[...] = x * n
    nrm_ref[...] = n


def _tc_encode(kid2, other, d0, d1, ktab8):
    return pl.pallas_call(
        _encode_body,
        grid=(N // R,),
        in_specs=[
            pl.BlockSpec((R, 1), lambda i: (i, 0)),
            pl.BlockSpec((R, 5), lambda i: (i, 0)),
            pl.BlockSpec((R, 1), lambda i: (i, 0)),
            pl.BlockSpec((R, 1), lambda i: (i, 0)),
            pl.BlockSpec((8, 8), lambda i: (0, 0)),
        ],
        out_specs=[
            pl.BlockSpec((R, 16), lambda i: (i, 0)),
            pl.BlockSpec((R, 1), lambda i: (i, 0)),
        ],
        out_shape=[
            jax.ShapeDtypeStruct((N, 16), _f32),
            jax.ShapeDtypeStruct((N, 1), _f32),
        ],
    )(kid2, other, d0, d1, ktab8)


def _combine0_body(s2, z0, nrm, w, b, z4_ref):
    n = nrm[...]
    sblk = s2[...]
    agg = n * (sblk[0] + sblk[1] + z0[...])
    h = jnp.maximum(jnp.dot(agg, w[...], preferred_element_type=_f32, precision=lax.Precision.HIGHEST) + b[...], 0.0)
    z = h * n
    z4_ref[...] = jnp.stack([z[:, :16], z[:, 16:32], z[:, 32:48], z[:, 48:]],
                            axis=0)


def _tc_combine0(s2, z0, nrm, w0p, b0):
    return pl.pallas_call(
        _combine0_body,
        grid=(N // R,),
        in_specs=[
            pl.BlockSpec((2, R, 16), lambda i: (0, i, 0)),
            pl.BlockSpec((R, 16), lambda i: (i, 0)),
            pl.BlockSpec((R, 1), lambda i: (i, 0)),
            pl.BlockSpec((16, H), lambda i: (0, 0)),
            pl.BlockSpec((1, H), lambda i: (0, 0)),
        ],
        out_specs=pl.BlockSpec((4, R, 16), lambda i: (0, i, 0)),
        out_shape=jax.ShapeDtypeStruct((4, N, 16), _f32),
    )(s2, z0, nrm, w0p, b0)


def _combine12_body(s4, z4, nrm, w, b, z4_ref):
    n = nrm[...]
    sblk = s4[...]
    zblk = z4[...]
    ss = jnp.concatenate([sblk[0], sblk[1], sblk[2], sblk[3]], axis=1)
    z = jnp.concatenate([zblk[0], zblk[1], zblk[2], zblk[3]], axis=1)
    agg = n * (ss + z)
    h = jnp.maximum(jnp.dot(agg, w[...], preferred_element_type=_f32, precision=lax.Precision.HIGHEST) + b[...], 0.0)
    zo = h * n
    z4_ref[...] = jnp.stack([zo[:, :16], zo[:, 16:32], zo[:, 32:48],
                             zo[:, 48:]], axis=0)


def _tc_combine12(s4, z4, nrm, w, b):
    return pl.pallas_call(
        _combine12_body,
        grid=(N // R,),
        in_specs=[
            pl.BlockSpec((4, R, 16), lambda i: (0, i, 0)),
            pl.BlockSpec((4, R, 16), lambda i: (0, i, 0)),
            pl.BlockSpec((R, 1), lambda i: (i, 0)),
            pl.BlockSpec((H, H), lambda i: (0, 0)),
            pl.BlockSpec((1, H), lambda i: (0, 0)),
        ],
        out_specs=pl.BlockSpec((4, R, 16), lambda i: (0, i, 0)),
        out_shape=jax.ShapeDtypeStruct((4, N, 16), _f32),
    )(s4, z4, nrm, w, b)


def _final_body(s4, z4, nrm, w, b, h_ref, acc_ref):
    n = nrm[...]
    sblk = s4[...]
    zblk = z4[...]
    ss = jnp.concatenate([sblk[0], sblk[1], sblk[2], sblk[3]], axis=1)
    z = jnp.concatenate([zblk[0], zblk[1], zblk[2], zblk[3]], axis=1)
    agg = n * (ss + z)
    h = jnp.maximum(jnp.dot(agg, w[...], preferred_element_type=_f32, precision=lax.Precision.HIGHEST) + b[...], 0.0)
    h_ref[...] = h

    @pl.when(pl.program_id(0) == 0)
    def _():
        acc_ref[...] = jnp.zeros((8, H), _f32)

    acc_ref[...] += jnp.sum(h.reshape(R // 8, 8, H), axis=0)


def _tc_final(s4, z4, nrm, w, b):
    return pl.pallas_call(
        _final_body,
        grid=(N // R,),
        in_specs=[
            pl.BlockSpec((4, R, 16), lambda i: (0, i, 0)),
            pl.BlockSpec((4, R, 16), lambda i: (0, i, 0)),
            pl.BlockSpec((R, 1), lambda i: (i, 0)),
            pl.BlockSpec((H, H), lambda i: (0, 0)),
            pl.BlockSpec((1, H), lambda i: (0, 0)),
        ],
        out_specs=[
            pl.BlockSpec((R, H), lambda i: (i, 0)),
            pl.BlockSpec((8, H), lambda i: (0, 0)),
        ],
        out_shape=[
            jax.ShapeDtypeStruct((N, H), _f32),
            jax.ShapeDtypeStruct((8, H), _f32),
        ],
    )(s4, z4, nrm, w, b)


def _score_body(u, v, acc, wg, bg, wa, wb, wc, bc1, w2, bc2, out_ref):
    tot = jnp.sum(acc[...], axis=0, keepdims=True) * (1.0 / N)
    g = jnp.dot(tot, wg[...], preferred_element_type=_f32, precision=lax.Precision.HIGHEST) + bg[...]
    base = jnp.dot(g, wc[...], preferred_element_type=_f32, precision=lax.Precision.HIGHEST) + bc1[...]
    hid = (jnp.dot(u[...], wa[...], preferred_element_type=_f32, precision=lax.Precision.HIGHEST)
           + jnp.dot(v[...], wb[...], preferred_element_type=_f32, precision=lax.Precision.HIGHEST) + base)
    hid = jnp.maximum(hid, 0.0)
    lg = jnp.sum(hid * w2[...], axis=1, keepdims=True) + bc2[...]
    out_ref[...] = lg


def _tc_score(u, v, acc, wg, bg, wa, wb, wc, bc1, w2row, bc2):
    whole = lambda shp: pl.BlockSpec(shp, lambda: (0, 0))
    return pl.pallas_call(
        _score_body,
        in_specs=[
            whole((C, H)), whole((C, H)), whole((8, H)),
            whole((H, H)), whole((1, H)),
            whole((H, H)), whole((H, H)), whole((H, H)),
            whole((1, H)), whole((1, H)), whole((1, 1)),
        ],
        out_specs=whole((C, 1)),
        out_shape=jax.ShapeDtypeStruct((C, 1), _f32),
    )(u, v, acc, wg, bg, wa, wb, wc, bc1, w2row, bc2)


def kernel(kind_ids, other_feats, edge_index, cand_u, cand_v, kind_table,
           W0, b0, W1, b1, W2, b2, Wg, bg, Wc1, bc1, Wc2, bc2):
    ei = edge_index.astype(_i32)
    src, dst = ei[0], ei[1]
    deg2 = _sc_degree(dst)
    ktab8 = jnp.concatenate([kind_table, jnp.zeros((2, 8), _f32)], axis=0)
    z0, nrm = _tc_encode(
        kind_ids.astype(_i32).reshape(N, 1), other_feats,
        deg2[0].reshape(NPAD, 1)[:N], deg2[1].reshape(NPAD, 1)[:N], ktab8)
    s0 = _sc_segsum16(src, dst, z0)
    w0p = jnp.concatenate([W0, jnp.zeros((3, H), _f32)], axis=0)
    z1 = _tc_combine0(s0, z0, nrm, w0p, b0.reshape(1, H))
    s1 = _sc_segsum64(src, dst, z1)
    z2 = _tc_combine12(s1, z1, nrm, W1, b1.reshape(1, H))
    s2 = _sc_segsum64(src, dst, z2)
    h3, acc = _tc_final(s2, z2, nrm, W2, b2.reshape(1, H))
    cuv = jnp.stack([cand_u.astype(_i32), cand_v.astype(_i32)], axis=0)
    uvr = _sc_cand_gather(h3, cuv)
    ur, vr = uvr[0], uvr[1]
    lg = _tc_score(ur, vr, acc, Wg, bg.reshape(1, H),
                   Wc1[:H], Wc1[H:2 * H], Wc1[2 * H:], bc1.reshape(1, H),
                   Wc2.reshape(1, H), bc2.reshape(1, 1))
    return lg.reshape(C)
